# baseline (speedup 1.0000x reference)
import functools
import math

import jax
import jax.numpy as jnp
from jax import lax
from jax.experimental import pallas as pl
from jax.experimental.pallas import tpu as pltpu
from jax.experimental.pallas import tpu_sc as plsc

DIMS = [32, 64, 128, 256, 512]
K = 16
LATENT = 256
RATIO = 0.25
BN_EPS = 1e-05

_LANES = 128
_INT_MAX = 2147483647


def _fps_body(n_sample, n, r, px_ref, py_ref, pz_ref, sx_ref, sy_ref, sz_ref,
              sel_ref, qx_ref, qy_ref, qz_ref, dmin_ref):
    row = jax.lax.broadcasted_iota(jnp.int32, (r, _LANES), 0)
    col = jax.lax.broadcasted_iota(jnp.int32, (r, _LANES), 1)
    flat = row * _LANES + col
    valid = flat < n

    sel_ref[0] = 0
    x0 = sx_ref[0]
    y0 = sy_ref[0]
    z0 = sz_ref[0]
    qx_ref[0] = x0
    qy_ref[0] = y0
    qz_ref[0] = z0
    dmin_ref[:] = jnp.where(valid, jnp.inf, -jnp.inf).astype(jnp.float32)

    def body(i, carry):
        lx, ly, lz = carry
        d = ((px_ref[:] - lx) ** 2 + (py_ref[:] - ly) ** 2
             + (pz_ref[:] - lz) ** 2)
        dm = jnp.minimum(dmin_ref[:], d)
        dmin_ref[:] = dm
        m = jnp.max(dm)
        sel_i = jnp.min(jnp.where(dm == m, flat, _INT_MAX))
        nx = sx_ref[sel_i]
        ny = sy_ref[sel_i]
        nz = sz_ref[sel_i]
        sel_ref[i] = sel_i
        qx_ref[i] = nx
        qy_ref[i] = ny
        qz_ref[i] = nz
        return (nx, ny, nz)

    jax.lax.fori_loop(1, n_sample, body, (x0, y0, z0))


def _fps_pallas(pxp, pyp, pzp, n, n_sample):
    r = pxp.shape[0]
    kern = functools.partial(_fps_body, n_sample, n, r)
    out = pl.pallas_call(
        kern,
        out_shape=(
            jax.ShapeDtypeStruct((n_sample,), jnp.int32),
            jax.ShapeDtypeStruct((n_sample,), jnp.float32),
            jax.ShapeDtypeStruct((n_sample,), jnp.float32),
            jax.ShapeDtypeStruct((n_sample,), jnp.float32),
        ),
        in_specs=[
            pl.BlockSpec((r, _LANES), lambda: (0, 0)),
            pl.BlockSpec((r, _LANES), lambda: (0, 0)),
            pl.BlockSpec((r, _LANES), lambda: (0, 0)),
            pl.BlockSpec(memory_space=pltpu.SMEM),
            pl.BlockSpec(memory_space=pltpu.SMEM),
            pl.BlockSpec(memory_space=pltpu.SMEM),
        ],
        out_specs=(
            pl.BlockSpec(memory_space=pltpu.SMEM),
            pl.BlockSpec(memory_space=pltpu.SMEM),
            pl.BlockSpec(memory_space=pltpu.SMEM),
            pl.BlockSpec(memory_space=pltpu.SMEM),
        ),
        scratch_shapes=[pltpu.VMEM((r, _LANES), jnp.float32)],
    )(pxp, pyp, pzp, pxp.reshape(-1), pyp.reshape(-1), pzp.reshape(-1))
    return out


def _planes(pos):
    n = pos.shape[0]
    r = max(8, ((n + _LANES - 1) // _LANES + 7) // 8 * 8)
    npad = r * _LANES
    p = jnp.pad(pos, ((0, npad - n), (0, 0)), constant_values=1e6)
    return (p[:, 0].reshape(r, _LANES), p[:, 1].reshape(r, _LANES),
            p[:, 2].reshape(r, _LANES))


def _fps(pos, n_sample):
    pxp, pyp, pzp = _planes(pos)
    sel, qx, qy, qz = _fps_pallas(pxp, pyp, pzp, pos.shape[0], n_sample)
    return sel, jnp.stack([qx, qy, qz], axis=1)


def _linear(p, x):
    y = x @ p['w']
    if 'b' in p:
        y = y + p['b']
    return y


def _mlp(ps, x):
    for p in ps:
        x = jax.nn.relu(_linear(p, x))
    return x


def _bn(p, x):
    import numpy as np
    return x * (p['g'] / np.float32(np.sqrt(1.0 + BN_EPS))) + p['beta']


_NW = 32
_SCL = 16
_G = 16


def _knn_sc_kern(qp, npad, n_true, exclude_self,
                 qx_hbm, qy_hbm, qz_hbm, kx_hbm, ky_hbm, kz_hbm, out_hbm,
                 kxv, kyv, kzv, qxv, qyv, qzv, outv):
    w = qp // _NW
    wid = lax.axis_index("s") * 2 + lax.axis_index("c")
    base = wid * w
    pltpu.sync_copy(kx_hbm, kxv)
    pltpu.sync_copy(ky_hbm, kyv)
    pltpu.sync_copy(kz_hbm, kzv)
    pltpu.sync_copy(qx_hbm.at[pl.ds(base, w)], qxv)
    pltpu.sync_copy(qy_hbm.at[pl.ds(base, w)], qyv)
    pltpu.sync_copy(qz_hbm.at[pl.ds(base, w)], qzv)

    iota = lax.iota(jnp.int32, _SCL)
    nj = npad // _SCL
    inf = jnp.float32(jnp.inf)

    def row_group(gi, _):
        r0 = gi * _G
        qxvec = qxv[pl.ds(r0, _SCL)]
        qyvec = qyv[pl.ds(r0, _SCL)]
        qzvec = qzv[pl.ds(r0, _SCL)]
        qs = [(qxvec[g], qyvec[g], qzvec[g]) for g in range(_G)]

        def jbody(j, carry):
            svals, sidxs = carry
            j16 = j * _SCL
            kx = kxv[pl.ds(j16, _SCL)]
            ky = kyv[pl.ds(j16, _SCL)]
            kz = kzv[pl.ds(j16, _SCL)]
            col = iota + j16
            pad_mask = col < n_true
            nv_list = []
            ni_list = []
            for g in range(_G):
                qx, qy, qz = qs[g]
                dx = kx - qx
                dy = ky - qy
                dz = kz - qz
                d = dx * dx + dy * dy + dz * dz
                if exclude_self:
                    gid = base + r0 + g
                    d = jnp.where(col == gid, d + jnp.float32(1e10), d)
                d = jnp.where(pad_mask, d, inf)
                dv, di = plsc.sort_key_val(d, col)
                sv, si = plsc.sort_key_val(svals[g], sidxs[g])
                dvr = lax.rev(dv, (0,))
                dir_ = lax.rev(di, (0,))
                m = dvr < sv
                nv_list.append(jnp.where(m, dvr, sv))
                ni_list.append(jnp.where(m, dir_, si))
            return (tuple(nv_list), tuple(ni_list))

        init = (tuple(jnp.full((_SCL,), inf, jnp.float32) for _ in range(_G)),
                tuple(jnp.zeros((_SCL,), jnp.int32) for _ in range(_G)))
        svals, sidxs = lax.fori_loop(0, nj, jbody, init)
        for g in range(_G):
            _, si = plsc.sort_key_val(svals[g], sidxs[g])
            outv[pl.ds((r0 + g) * _SCL, _SCL)] = si
        return 0

    lax.fori_loop(0, w // _G, row_group, 0)
    pltpu.sync_copy(outv, out_hbm.at[pl.ds(base * _SCL, w * _SCL)])


@functools.lru_cache(maxsize=None)
def _knn_sc_call(qp, npad, n_true, exclude_self):
    w = qp // _NW
    mesh = plsc.VectorSubcoreMesh(core_axis_name="c", subcore_axis_name="s",
                                  num_cores=2, num_subcores=16)
    body = functools.partial(_knn_sc_kern, qp, npad, n_true, exclude_self)
    return pl.kernel(
        body,
        out_type=jax.ShapeDtypeStruct((qp * _SCL,), jnp.int32),
        mesh=mesh,
        scratch_types=[
            pltpu.VMEM((npad,), jnp.float32),
            pltpu.VMEM((npad,), jnp.float32),
            pltpu.VMEM((npad,), jnp.float32),
            pltpu.VMEM((w,), jnp.float32),
            pltpu.VMEM((w,), jnp.float32),
            pltpu.VMEM((w,), jnp.float32),
            pltpu.VMEM((w * _SCL,), jnp.int32),
        ],
        name=f"knn_sc_q{qp}_n{npad}",
        compiler_params=pltpu.CompilerParams(needs_layout_passes=False),
    )


def _knn(query, keys, k, exclude_self=False):
    q = query.shape[0]
    n = keys.shape[0]
    w = ((q + _NW - 1) // _NW + _G - 1) // _G * _G
    qp = w * _NW
    npad = (n + _SCL - 1) // _SCL * _SCL
    qpad = jnp.pad(query, ((0, qp - q), (0, 0)), constant_values=1e4)
    kpad = jnp.pad(keys, ((0, npad - n), (0, 0)), constant_values=1e4)
    fn = _knn_sc_call(qp, npad, n, bool(exclude_self))
    out = fn(qpad[:, 0], qpad[:, 1], qpad[:, 2],
             kpad[:, 0], kpad[:, 1], kpad[:, 2])
    return out.reshape(qp, _SCL)[:q]


def _conv(p, x, pos, nbr):
    n = x.shape[0]
    nbr_full = jnp.concatenate([nbr, jnp.arange(n, dtype=nbr.dtype)[:, None]],
                               axis=1)
    alpha_dst = _linear(p['conv_dst'], x)
    alpha_src = _linear(p['conv_src'], x)[nbr_full]
    xj = _linear(p['conv_lin'], x)[nbr_full]
    rel = pos[:, None, :] - pos[nbr_full]
    delta = _mlp(p['pos_nn'], rel)
    alpha = _mlp(p['attn_nn'], alpha_dst[:, None, :] - alpha_src + delta)
    alpha = jax.nn.softmax(alpha, axis=1)
    return jnp.sum(alpha * (xj + delta), axis=1)


def _tblock(p, x, pos, nbr):
    x = jax.nn.relu(_linear(p['lin_in'], x))
    x = _conv(p, x, pos, nbr)
    x = jax.nn.relu(_linear(p['lin_out'], x))
    return x


def _tblock_const(p, x, pos, nbr):
    n = pos.shape[0]
    x = jax.nn.relu(_linear(p['lin_in'], x))
    alpha_dst = _linear(p['conv_dst'], x)
    alpha_src_row = _linear(p['conv_src'], x)[0]
    xj_row = _linear(p['conv_lin'], x)[0]
    nbr_full = jnp.concatenate([nbr, jnp.arange(n, dtype=nbr.dtype)[:, None]],
                               axis=1)
    rel = pos[:, None, :] - pos[nbr_full]
    delta = _mlp(p['pos_nn'], rel)
    alpha = _mlp(p['attn_nn'], alpha_dst[:, None, :] - alpha_src_row + delta)
    alpha = jax.nn.softmax(alpha, axis=1)
    out = jnp.sum(alpha * (xj_row + delta), axis=1)
    return jax.nn.relu(_linear(p['lin_out'], out))


def kernel(pos, batch, params):
    n = pos.shape[0]
    x = jnp.ones((n, 1), dtype=pos.dtype)
    x = jax.nn.relu(_bn(params['mlp_input']['bn'],
                        _linear(params['mlp_input']['lin'], x)))
    nbr = _knn(pos, pos, K, exclude_self=True)
    x = _tblock_const(params['tb_in'], x, pos, nbr)
    cur = n
    for i in range(4):
        n_sub = int(math.ceil(RATIO * cur))
        ids, pos_sub = _fps(pos, n_sub)
        nbr_td = _knn(pos_sub, pos, K, exclude_self=False)
        h = jax.nn.relu(_bn(params['td'][i]['bn'],
                            _linear(params['td'][i]['lin'], x)))
        x = jnp.max(h[nbr_td], axis=1)
        pos = pos_sub
        batch = batch[ids]
        nbr = _knn(pos, pos, K, exclude_self=True)
        x = _tblock(params['tb'][i], x, pos, nbr)
        cur = n_sub
    summed = jax.ops.segment_sum(x, batch, num_segments=1)
    counts = jax.ops.segment_sum(jnp.ones((x.shape[0], 1), x.dtype), batch,
                                 num_segments=1)
    out = summed / counts
    out = jax.nn.relu(_bn(params['lin_out']['bn'],
                          _linear(params['lin_out']['lin'], out)))
    return out


# speedup vs baseline: 1.1229x; 1.1229x over previous
import functools
import math

import jax
import jax.numpy as jnp
from jax import lax
from jax.experimental import pallas as pl
from jax.experimental.pallas import tpu as pltpu
from jax.experimental.pallas import tpu_sc as plsc

DIMS = [32, 64, 128, 256, 512]
K = 16
LATENT = 256
RATIO = 0.25
BN_EPS = 1e-05

_LANES = 128
_INT_MAX = 2147483647


def _fps_body(n_sample, n, r, px_ref, py_ref, pz_ref, sx_ref, sy_ref, sz_ref,
              sel_ref, qx_ref, qy_ref, qz_ref, dmin_ref):
    row = jax.lax.broadcasted_iota(jnp.int32, (r, _LANES), 0)
    col = jax.lax.broadcasted_iota(jnp.int32, (r, _LANES), 1)
    flat = row * _LANES + col
    valid = flat < n

    sel_ref[0] = 0
    x0 = sx_ref[0]
    y0 = sy_ref[0]
    z0 = sz_ref[0]
    qx_ref[0] = x0
    qy_ref[0] = y0
    qz_ref[0] = z0
    dmin_ref[:] = jnp.where(valid, jnp.inf, -jnp.inf).astype(jnp.float32)

    def body(i, carry):
        lx, ly, lz = carry
        d = ((px_ref[:] - lx) ** 2 + (py_ref[:] - ly) ** 2
             + (pz_ref[:] - lz) ** 2)
        dm = jnp.minimum(dmin_ref[:], d)
        dmin_ref[:] = dm
        m = jnp.max(dm)
        sel_i = jnp.min(jnp.where(dm == m, flat, _INT_MAX))
        nx = sx_ref[sel_i]
        ny = sy_ref[sel_i]
        nz = sz_ref[sel_i]
        sel_ref[i] = sel_i
        qx_ref[i] = nx
        qy_ref[i] = ny
        qz_ref[i] = nz
        return (nx, ny, nz)

    jax.lax.fori_loop(1, n_sample, body, (x0, y0, z0))


def _fps_pallas(pxp, pyp, pzp, n, n_sample):
    r = pxp.shape[0]
    kern = functools.partial(_fps_body, n_sample, n, r)
    out = pl.pallas_call(
        kern,
        out_shape=(
            jax.ShapeDtypeStruct((n_sample,), jnp.int32),
            jax.ShapeDtypeStruct((n_sample,), jnp.float32),
            jax.ShapeDtypeStruct((n_sample,), jnp.float32),
            jax.ShapeDtypeStruct((n_sample,), jnp.float32),
        ),
        in_specs=[
            pl.BlockSpec((r, _LANES), lambda: (0, 0)),
            pl.BlockSpec((r, _LANES), lambda: (0, 0)),
            pl.BlockSpec((r, _LANES), lambda: (0, 0)),
            pl.BlockSpec(memory_space=pltpu.SMEM),
            pl.BlockSpec(memory_space=pltpu.SMEM),
            pl.BlockSpec(memory_space=pltpu.SMEM),
        ],
        out_specs=(
            pl.BlockSpec(memory_space=pltpu.SMEM),
            pl.BlockSpec(memory_space=pltpu.SMEM),
            pl.BlockSpec(memory_space=pltpu.SMEM),
            pl.BlockSpec(memory_space=pltpu.SMEM),
        ),
        scratch_shapes=[pltpu.VMEM((r, _LANES), jnp.float32)],
    )(pxp, pyp, pzp, pxp.reshape(-1), pyp.reshape(-1), pzp.reshape(-1))
    return out


def _planes(pos):
    n = pos.shape[0]
    r = max(8, ((n + _LANES - 1) // _LANES + 7) // 8 * 8)
    npad = r * _LANES
    p = jnp.pad(pos, ((0, npad - n), (0, 0)), constant_values=1e6)
    return (p[:, 0].reshape(r, _LANES), p[:, 1].reshape(r, _LANES),
            p[:, 2].reshape(r, _LANES))


def _fps(pos, n_sample):
    pxp, pyp, pzp = _planes(pos)
    sel, qx, qy, qz = _fps_pallas(pxp, pyp, pzp, pos.shape[0], n_sample)
    return sel, jnp.stack([qx, qy, qz], axis=1)


def _linear(p, x):
    y = x @ p['w']
    if 'b' in p:
        y = y + p['b']
    return y


def _mlp(ps, x):
    for p in ps:
        x = jax.nn.relu(_linear(p, x))
    return x


def _bn(p, x):
    import numpy as np
    return x * (p['g'] / np.float32(np.sqrt(1.0 + BN_EPS))) + p['beta']


_NW = 32
_SCL = 16
_G = 16


def _knn_sc_kern(qp, npad, n_true, exclude_self,
                 qx_hbm, qy_hbm, qz_hbm, kx_hbm, ky_hbm, kz_hbm, out_hbm,
                 kxv, kyv, kzv, qxv, qyv, qzv, outv):
    w = qp // _NW
    wid = lax.axis_index("s") * 2 + lax.axis_index("c")
    base = wid * w
    pltpu.sync_copy(kx_hbm, kxv)
    pltpu.sync_copy(ky_hbm, kyv)
    pltpu.sync_copy(kz_hbm, kzv)
    pltpu.sync_copy(qx_hbm.at[pl.ds(base, w)], qxv)
    pltpu.sync_copy(qy_hbm.at[pl.ds(base, w)], qyv)
    pltpu.sync_copy(qz_hbm.at[pl.ds(base, w)], qzv)

    iota = lax.iota(jnp.int32, _SCL)
    nj = npad // _SCL
    inf = jnp.float32(jnp.inf)

    def row_group(gi, _):
        r0 = gi * _G
        qxvec = qxv[pl.ds(r0, _SCL)]
        qyvec = qyv[pl.ds(r0, _SCL)]
        qzvec = qzv[pl.ds(r0, _SCL)]
        qs = [(qxvec[g], qyvec[g], qzvec[g]) for g in range(_G)]

        def jbody(j, carry):
            svals, sidxs = carry
            j16 = j * _SCL
            kx = kxv[pl.ds(j16, _SCL)]
            ky = kyv[pl.ds(j16, _SCL)]
            kz = kzv[pl.ds(j16, _SCL)]
            col = iota + j16
            nv_list = []
            ni_list = []
            for g in range(_G):
                qx, qy, qz = qs[g]
                dx = kx - qx
                dy = ky - qy
                dz = kz - qz
                d = dx * dx + dy * dy + dz * dz
                if exclude_self:
                    gid = base + r0 + g
                    d = jnp.where(col == gid, d + jnp.float32(1e10), d)
                dv, di = plsc.sort_key_val(d, col)
                m = dv < svals[g]
                u = jnp.where(m, dv, svals[g])
                ui = jnp.where(m, di, sidxs[g])
                nv, ni = plsc.sort_key_val(u, ui, descending=True)
                nv_list.append(nv)
                ni_list.append(ni)
            return (tuple(nv_list), tuple(ni_list))

        init = (tuple(jnp.full((_SCL,), inf, jnp.float32) for _ in range(_G)),
                tuple(jnp.zeros((_SCL,), jnp.int32) for _ in range(_G)))
        svals, sidxs = lax.fori_loop(0, nj, jbody, init)
        for g in range(_G):
            outv[pl.ds((r0 + g) * _SCL, _SCL)] = sidxs[g]
        return 0

    lax.fori_loop(0, w // _G, row_group, 0)
    pltpu.sync_copy(outv, out_hbm.at[pl.ds(base * _SCL, w * _SCL)])


@functools.lru_cache(maxsize=None)
def _knn_sc_call(qp, npad, n_true, exclude_self):
    w = qp // _NW
    mesh = plsc.VectorSubcoreMesh(core_axis_name="c", subcore_axis_name="s",
                                  num_cores=2, num_subcores=16)
    body = functools.partial(_knn_sc_kern, qp, npad, n_true, exclude_self)
    return pl.kernel(
        body,
        out_type=jax.ShapeDtypeStruct((qp * _SCL,), jnp.int32),
        mesh=mesh,
        scratch_types=[
            pltpu.VMEM((npad,), jnp.float32),
            pltpu.VMEM((npad,), jnp.float32),
            pltpu.VMEM((npad,), jnp.float32),
            pltpu.VMEM((w,), jnp.float32),
            pltpu.VMEM((w,), jnp.float32),
            pltpu.VMEM((w,), jnp.float32),
            pltpu.VMEM((w * _SCL,), jnp.int32),
        ],
        name=f"knn_sc_q{qp}_n{npad}",
        compiler_params=pltpu.CompilerParams(needs_layout_passes=False),
    )


def _knn(query, keys, k, exclude_self=False):
    q = query.shape[0]
    n = keys.shape[0]
    w = ((q + _NW - 1) // _NW + _G - 1) // _G * _G
    qp = w * _NW
    npad = (n + _SCL - 1) // _SCL * _SCL
    qpad = jnp.pad(query, ((0, qp - q), (0, 0)), constant_values=1e4)
    kpad = jnp.pad(keys, ((0, npad - n), (0, 0)), constant_values=1e4)
    fn = _knn_sc_call(qp, npad, n, bool(exclude_self))
    out = fn(qpad[:, 0], qpad[:, 1], qpad[:, 2],
             kpad[:, 0], kpad[:, 1], kpad[:, 2])
    return out.reshape(qp, _SCL)[:q]


def _conv(p, x, pos, nbr):
    n = x.shape[0]
    nbr_full = jnp.concatenate([nbr, jnp.arange(n, dtype=nbr.dtype)[:, None]],
                               axis=1)
    alpha_dst = _linear(p['conv_dst'], x)
    alpha_src = _linear(p['conv_src'], x)[nbr_full]
    xj = _linear(p['conv_lin'], x)[nbr_full]
    rel = pos[:, None, :] - pos[nbr_full]
    delta = _mlp(p['pos_nn'], rel)
    alpha = _mlp(p['attn_nn'], alpha_dst[:, None, :] - alpha_src + delta)
    alpha = jax.nn.softmax(alpha, axis=1)
    return jnp.sum(alpha * (xj + delta), axis=1)


def _tblock(p, x, pos, nbr):
    x = jax.nn.relu(_linear(p['lin_in'], x))
    x = _conv(p, x, pos, nbr)
    x = jax.nn.relu(_linear(p['lin_out'], x))
    return x


def _tblock_const(p, x, pos, nbr):
    n = pos.shape[0]
    x = jax.nn.relu(_linear(p['lin_in'], x))
    alpha_dst = _linear(p['conv_dst'], x)
    alpha_src_row = _linear(p['conv_src'], x)[0]
    xj_row = _linear(p['conv_lin'], x)[0]
    nbr_full = jnp.concatenate([nbr, jnp.arange(n, dtype=nbr.dtype)[:, None]],
                               axis=1)
    rel = pos[:, None, :] - pos[nbr_full]
    delta = _mlp(p['pos_nn'], rel)
    alpha = _mlp(p['attn_nn'], alpha_dst[:, None, :] - alpha_src_row + delta)
    alpha = jax.nn.softmax(alpha, axis=1)
    out = jnp.sum(alpha * (xj_row + delta), axis=1)
    return jax.nn.relu(_linear(p['lin_out'], out))


def kernel(pos, batch, params):
    n = pos.shape[0]
    x = jnp.ones((n, 1), dtype=pos.dtype)
    x = jax.nn.relu(_bn(params['mlp_input']['bn'],
                        _linear(params['mlp_input']['lin'], x)))
    nbr = _knn(pos, pos, K, exclude_self=True)
    x = _tblock_const(params['tb_in'], x, pos, nbr)
    cur = n
    for i in range(4):
        n_sub = int(math.ceil(RATIO * cur))
        ids, pos_sub = _fps(pos, n_sub)
        nbr_td = _knn(pos_sub, pos, K, exclude_self=False)
        h = jax.nn.relu(_bn(params['td'][i]['bn'],
                            _linear(params['td'][i]['lin'], x)))
        x = jnp.max(h[nbr_td], axis=1)
        pos = pos_sub
        batch = batch[ids]
        nbr = _knn(pos, pos, K, exclude_self=True)
        x = _tblock(params['tb'][i], x, pos, nbr)
        cur = n_sub
    summed = jax.ops.segment_sum(x, batch, num_segments=1)
    counts = jax.ops.segment_sum(jnp.ones((x.shape[0], 1), x.dtype), batch,
                                 num_segments=1)
    out = summed / counts
    out = jax.nn.relu(_bn(params['lin_out']['bn'],
                          _linear(params['lin_out']['lin'], out)))
    return out
